# baseline (speedup 1.0000x reference)
import functools

import jax
import jax.numpy as jnp
from jax import lax
from jax.experimental import pallas as pl
from jax.experimental.pallas import tpu as pltpu
from jax.experimental.pallas import tpu_sc as plsc

_TOPK = 8
_NUM_EXPERTS = 64
_BLOCK_ROWS = 1024
_NW = 32
_LANES = 16


def _score_kernel(x_ref, e_ref, b_ref, s_ref, st_ref):
    logits_t = jax.lax.dot_general(
        e_ref[...], x_ref[...],
        dimension_numbers=(((1,), (1,)), ((), ())),
        preferred_element_type=jnp.float32,
    )
    scores_t = jax.nn.sigmoid(logits_t) + b_ref[...]
    s_ref[...] = scores_t.T
    st_ref[0] = scores_t


def _sc_topk_kernel(st_hbm, wt_hbm, it_hbm, st_v, wv_v, iv_v, rpw):
    wid = lax.axis_index("s") * 2 + lax.axis_index("c")
    pltpu.sync_copy(st_hbm.at[wid], st_v)
    neg_inf = jnp.float32(-jnp.inf)

    def group(g, carry):
        base = g * _LANES
        vals = [st_v[e, pl.ds(base, _LANES)] for e in range(_NUM_EXPERTS)]
        total = None
        tops = []
        for _ in range(_TOPK):
            m = vals[0]
            for e in range(1, _NUM_EXPERTS):
                m = jnp.maximum(m, vals[e])
            idx = jnp.full((_LANES,), _NUM_EXPERTS, jnp.int32)
            for e in range(_NUM_EXPERTS - 1, -1, -1):
                idx = jnp.where(vals[e] == m, e, idx)
            for e in range(_NUM_EXPERTS):
                vals[e] = jnp.where(idx == e, neg_inf, vals[e])
            tops.append((m, idx))
            total = m if total is None else total + m
        for k, (m, idx) in enumerate(tops):
            wv_v[k, pl.ds(base, _LANES)] = m / total
            iv_v[k, pl.ds(base, _LANES)] = idx
        return carry

    lax.fori_loop(0, rpw // _LANES, group, 0)
    pltpu.sync_copy(wv_v, wt_hbm.at[wid])
    pltpu.sync_copy(iv_v, it_hbm.at[wid])


@jax.jit
def kernel(x, expert_embeddings, bias):
    n_rows, n_cols = x.shape
    n_exp = expert_embeddings.shape[0]
    rpw = n_rows // _NW
    grid = (n_rows // _BLOCK_ROWS,)
    bias2d = bias.reshape(n_exp, 1)
    scores, st = pl.pallas_call(
        _score_kernel,
        grid=grid,
        in_specs=[
            pl.BlockSpec((_BLOCK_ROWS, n_cols), lambda i: (i, 0)),
            pl.BlockSpec((n_exp, n_cols), lambda i: (0, 0)),
            pl.BlockSpec((n_exp, 1), lambda i: (0, 0)),
        ],
        out_specs=[
            pl.BlockSpec((_BLOCK_ROWS, n_exp), lambda i: (i, 0)),
            pl.BlockSpec((1, n_exp, _BLOCK_ROWS), lambda i: (i, 0, 0)),
        ],
        out_shape=[
            jax.ShapeDtypeStruct((n_rows, n_exp), jnp.float32),
            jax.ShapeDtypeStruct((_NW, n_exp, rpw), jnp.float32),
        ],
    )(x, expert_embeddings, bias2d)

    mesh = plsc.VectorSubcoreMesh(core_axis_name="c", subcore_axis_name="s")
    wt, it = pl.kernel(
        functools.partial(_sc_topk_kernel, rpw=rpw),
        mesh=mesh,
        out_type=[
            jax.ShapeDtypeStruct((_NW, _TOPK, rpw), jnp.float32),
            jax.ShapeDtypeStruct((_NW, _TOPK, rpw), jnp.int32),
        ],
        scratch_types=[
            pltpu.VMEM((n_exp, rpw), jnp.float32),
            pltpu.VMEM((_TOPK, rpw), jnp.float32),
            pltpu.VMEM((_TOPK, rpw), jnp.int32),
        ],
    )(st)

    weights = wt.transpose(0, 2, 1).reshape(n_rows, _TOPK)
    indices = it.transpose(0, 2, 1).reshape(n_rows, _TOPK)
    return weights.astype(x.dtype), indices, scores
